# baseline (speedup 1.0000x reference)
import jax
import jax.numpy as jnp
from jax import lax
from jax.experimental import pallas as pl
from jax.experimental.pallas import tpu as pltpu
from jax.experimental.pallas import tpu_sc as plsc

_B = 128
_NV = 100000
_NVP = 100352
_A = _NVP // 4
_NWK = 16
_AW = _A // _NWK
_SUBW = 56
_NSUB = _AW // _SUBW
_S = _NWK * _NSUB
_BLKW = _SUBW * 128
_META = _S * 128
_CONT = 16
_NW = 16
_RPW = _B // _NW


def _fold_bytes(acc):
    m = jnp.int32(0xFF)
    return ((acc & m)
            + (lax.shift_right_logical(acc, 8) & m)
            + (lax.shift_right_logical(acc, 16) & m)
            + (lax.shift_right_logical(acc, 24) & m))


def _p1_body(words_ref, meta_ref, buf0, buf1, mloc, sem0, sem1):
    wid = lax.axis_index("c") * 16 + lax.axis_index("s")
    a0 = wid * _AW
    words_w = words_ref.bitcast(jnp.int32)
    bufs = [buf0, buf1]
    sems = [sem0, sem1]
    copies = [None, None]
    copies[0] = pltpu.async_copy(
        words_w.at[pl.ds(a0, _SUBW), :], buf0, sem0)

    for j in range(_NSUB):
        buf = bufs[j % 2]
        copies[j % 2].wait()
        if j + 1 < _NSUB:
            copies[(j + 1) % 2] = pltpu.async_copy(
                words_w.at[pl.ds(a0 + (j + 1) * _SUBW, _SUBW), :],
                bufs[(j + 1) % 2], sems[(j + 1) % 2])

        def row_body(a, accs, buf=buf):
            return tuple(accs[g] + buf[a, pl.ds(g * 16, 16)]
                         for g in range(8))

        accs = lax.fori_loop(
            0, _SUBW, row_body,
            tuple(jnp.zeros((16,), jnp.int32) for _ in range(8)))
        for g in range(8):
            mloc[pl.ds((g * _NSUB + j) * 16, 16)] = _fold_bytes(accs[g])

    for g in range(8):
        pltpu.sync_copy(
            mloc.at[pl.ds(g * _NSUB * 16, _NSUB * 16)],
            meta_ref.at[pl.ds(g * (_S * 16) + wid * (_NSUB * 16),
                              _NSUB * 16)])


def _p2_body(words_ref, meta_ref, u_ref, u01_ref, cont_ref, out_ref,
             mbuf, sbuf, ubuf, contbuf, resv, semm, sems):
    wid = lax.axis_index("c") * 16 + lax.axis_index("s")
    words_w = words_ref.bitcast(jnp.int32)
    r0 = wid * _RPW
    iota = lax.iota(jnp.int32, 16)
    gidx = wid // 2
    p0 = (wid % 2) * 8

    cm = pltpu.async_copy(
        meta_ref.at[pl.ds(gidx * (_S * 16), _S * 16)],
        mbuf.at[pl.ds(0, _S * 16)], semm)

    pltpu.sync_copy(u01_ref.at[pl.ds(r0 * _CONT, _RPW * _CONT)], contbuf)
    for q in range(_RPW):
        cv = contbuf[pl.ds(q * 16, 16)]
        contbuf[pl.ds(q * 16, 16)] = cv * jnp.float32(2.0) - jnp.float32(1.0)
    pltpu.sync_copy(contbuf, cont_ref.at[pl.ds(r0 * _CONT, _RPW * _CONT)])

    pltpu.sync_copy(u_ref.at[pl.ds(gidx * 16, 16)], ubuf)
    uv = ubuf[...]
    cm.wait()

    goff = gidx * 16

    def t_body(s, tot):
        return tot + mbuf[pl.ds(s * 16, 16)]

    tot = lax.fori_loop(0, _S, t_body, jnp.zeros((16,), jnp.int32))

    kv = (uv * tot.astype(jnp.float32)).astype(jnp.int32)
    kv = jnp.where((kv >= tot) | (tot <= 0), jnp.int32(-1), kv)

    def x_body(s, c):
        run, sv, rv = c
        mv = mbuf[pl.ds(s * 16, 16)]
        run2 = run + mv
        crossed = (run <= kv) & (run2 > kv) & (sv < 0)
        sv = jnp.where(crossed, s, sv)
        rv = jnp.where(crossed, run, rv)
        return (run2, sv, rv)

    _, sv, rv = lax.fori_loop(
        0, _S, x_body,
        (jnp.zeros((16,), jnp.int32), jnp.full((16,), -1, jnp.int32),
         jnp.zeros((16,), jnp.int32)))

    res = jnp.zeros((16,), jnp.int32)
    for q in range(_RPW):
        p = p0 + q
        s_star = jnp.sum(jnp.where(iota == p, sv, 0))
        rbef = jnp.sum(jnp.where(iota == p, rv, 0))
        kq = jnp.sum(jnp.where(iota == p, kv, 0))
        s_c = jnp.maximum(s_star, 0)

        pltpu.async_copy(words_w.at[pl.ds(s_c * _SUBW, _SUBW), :],
                         sbuf, sems).wait()

        def cnt(a):
            fv = _fold_bytes(sbuf[a, pl.ds(goff, 16)])
            return jnp.sum(jnp.where(iota == p, fv, 0))

        def w_cond(c):
            _, run2, t = c
            return run2 + t <= kq

        def w_body(c):
            a, run2, t = c
            a2 = a + 1
            return (a2, run2 + t, cnt(a2))

        a, run2, _ = lax.while_loop(
            w_cond, w_body, (jnp.int32(0), rbef, cnt(jnp.int32(0))))

        vw = sbuf[a, pl.ds(goff, 16)]
        wsc = jnp.sum(jnp.where(iota == p, vw, 0))
        m8 = jnp.int32(0xFF)
        b0 = wsc & m8
        b1 = lax.shift_right_logical(wsc, 8) & m8
        b2 = lax.shift_right_logical(wsc, 16) & m8
        b3 = lax.shift_right_logical(wsc, 24) & m8
        rl = kq - run2
        e1 = b0
        e2 = b0 + b1
        e3 = e2 + b2
        i_sel = jnp.where(
            (rl == 0) & (b0 > 0), 0,
            jnp.where((e1 == rl) & (b1 > 0), 1,
                      jnp.where((e2 == rl) & (b2 > 0), 2,
                                jnp.where((e3 == rl) & (b3 > 0), 3, 0))))
        idx = (s_c * _SUBW + a) * 4 + i_sel
        idx = jnp.where(s_star < 0, 0, idx)
        res = jnp.where(iota == q, idx, res)

    resv[...] = res
    pltpu.sync_copy(resv, out_ref.at[pl.ds(wid * 16, 16)])


_mesh = plsc.VectorSubcoreMesh(core_axis_name="c", subcore_axis_name="s",
                               num_cores=1)

_sc_p1 = pl.kernel(
    _p1_body,
    out_type=jax.ShapeDtypeStruct((_META,), jnp.int32),
    mesh=_mesh,
    scratch_types=[
        pltpu.VMEM((_SUBW, 128), jnp.int32),
        pltpu.VMEM((_SUBW, 128), jnp.int32),
        pltpu.VMEM((_NSUB * 128,), jnp.int32),
        pltpu.SemaphoreType.DMA,
        pltpu.SemaphoreType.DMA,
    ],
    compiler_params=pltpu.CompilerParams(needs_layout_passes=False),
)

_sc_p2 = pl.kernel(
    _p2_body,
    out_type=(jax.ShapeDtypeStruct((_B * _CONT,), jnp.float32),
              jax.ShapeDtypeStruct((_NW * 16,), jnp.int32)),
    mesh=_mesh,
    scratch_types=[
        pltpu.VMEM((_S * 16,), jnp.int32),
        pltpu.VMEM((_SUBW, 128), jnp.int32),
        pltpu.VMEM((16,), jnp.float32),
        pltpu.VMEM((_RPW * _CONT,), jnp.float32),
        pltpu.VMEM((16,), jnp.int32),
        pltpu.SemaphoreType.DMA,
        pltpu.SemaphoreType.DMA,
    ],
    compiler_params=pltpu.CompilerParams(needs_layout_passes=False),
)


def kernel(states, mask):
    del states
    key = jax.random.key(42)
    ka, kb = jax.random.split(key)
    u01 = jax.random.uniform(ka, (_B, _CONT), dtype=jnp.float32)
    u = jax.random.uniform(kb, (_B,), dtype=jnp.float32)

    mbytes = jnp.pad(mask.astype(jnp.uint8).T, ((0, _NVP - _NV), (0, 0)))

    meta = _sc_p1(mbytes)
    cont_flat, disc_flat = _sc_p2(mbytes, meta, u, u01.reshape(-1))
    cont = cont_flat.reshape(_B, _CONT)
    disc = disc_flat.reshape(_NW, 16)[:, :_RPW].reshape(_B)
    return cont, disc


# speedup vs baseline: 1.3308x; 1.3308x over previous
import jax
import jax.numpy as jnp
from jax import lax
from jax.experimental import pallas as pl
from jax.experimental.pallas import tpu as pltpu
from jax.experimental.pallas import tpu_sc as plsc

_B = 128
_NV = 100000
_NVP = 100352
_A = _NVP // 4
_AW = _A // 32
_SUBW = 56
_NSUB = _AW // _SUBW
_S = 32 * _NSUB
_BLKW = _SUBW * 128
_META = _S * 128
_CONT = 16
_NW = 32
_RPW = _B // _NW


def _fold_bytes(acc):
    m = jnp.int32(0xFF)
    return ((acc & m)
            + (lax.shift_right_logical(acc, 8) & m)
            + (lax.shift_right_logical(acc, 16) & m)
            + (lax.shift_right_logical(acc, 24) & m))


def _p1_body(words_ref, meta_ref, buf0, buf1, mloc, sem0, sem1):
    wid = lax.axis_index("c") * 16 + lax.axis_index("s")
    a0 = wid * _AW
    words_w = words_ref.bitcast(jnp.int32)
    bufs = [buf0, buf1]
    sems = [sem0, sem1]
    copies = [None, None]
    copies[0] = pltpu.async_copy(
        words_w.at[pl.ds(a0, _SUBW), :], buf0, sem0)

    for j in range(_NSUB):
        buf = bufs[j % 2]
        copies[j % 2].wait()
        if j + 1 < _NSUB:
            copies[(j + 1) % 2] = pltpu.async_copy(
                words_w.at[pl.ds(a0 + (j + 1) * _SUBW, _SUBW), :],
                bufs[(j + 1) % 2], sems[(j + 1) % 2])

        def row_body(a, accs, buf=buf):
            return tuple(accs[g] + buf[a, pl.ds(g * 16, 16)]
                         for g in range(8))

        accs = lax.fori_loop(
            0, _SUBW, row_body,
            tuple(jnp.zeros((16,), jnp.int32) for _ in range(8)))
        for g in range(8):
            mloc[pl.ds((g * _NSUB + j) * 16, 16)] = _fold_bytes(accs[g])

    for g in range(8):
        pltpu.sync_copy(
            mloc.at[pl.ds(g * _NSUB * 16, _NSUB * 16)],
            meta_ref.at[pl.ds(g * (_S * 16) + wid * (_NSUB * 16),
                              _NSUB * 16)])


def _p2_body(words_ref, meta_ref, u_ref, u01_ref, cont_ref, out_ref,
             mbuf, sbuf0, sbuf1, sbuf2, sbuf3, ubuf, contbuf, resv,
             semm, semb0, semb1, semb2, semb3):
    wid = lax.axis_index("c") * 16 + lax.axis_index("s")
    words_w = words_ref.bitcast(jnp.int32)
    r0 = wid * _RPW
    iota = lax.iota(jnp.int32, 16)
    gidx = wid // 4
    p0 = (wid % 4) * 4

    cm = pltpu.async_copy(
        meta_ref.at[pl.ds(gidx * (_S * 16), _S * 16)],
        mbuf.at[pl.ds(0, _S * 16)], semm)

    pltpu.sync_copy(u01_ref.at[pl.ds(r0 * _CONT, _RPW * _CONT)], contbuf)
    for q in range(_RPW):
        cv = contbuf[pl.ds(q * 16, 16)]
        contbuf[pl.ds(q * 16, 16)] = cv * jnp.float32(2.0) - jnp.float32(1.0)
    pltpu.sync_copy(contbuf, cont_ref.at[pl.ds(r0 * _CONT, _RPW * _CONT)])

    pltpu.sync_copy(u_ref.at[pl.ds(gidx * 16, 16)], ubuf)
    uv = ubuf[...]
    cm.wait()

    goff = gidx * 16

    def t_body(s, tot):
        return tot + mbuf[pl.ds(s * 16, 16)]

    tot = lax.fori_loop(0, _S, t_body, jnp.zeros((16,), jnp.int32))

    kv = (uv * tot.astype(jnp.float32)).astype(jnp.int32)
    kv = jnp.where((kv >= tot) | (tot <= 0), jnp.int32(-1), kv)

    def x_body(s, c):
        run, sv, rv = c
        mv = mbuf[pl.ds(s * 16, 16)]
        run2 = run + mv
        crossed = (run <= kv) & (run2 > kv) & (sv < 0)
        sv = jnp.where(crossed, s, sv)
        rv = jnp.where(crossed, run, rv)
        return (run2, sv, rv)

    _, sv, rv = lax.fori_loop(
        0, _S, x_body,
        (jnp.zeros((16,), jnp.int32), jnp.full((16,), -1, jnp.int32),
         jnp.zeros((16,), jnp.int32)))

    res = jnp.zeros((16,), jnp.int32)
    sbufs = [sbuf0, sbuf1, sbuf2, sbuf3]
    bsems = [semb0, semb1, semb2, semb3]
    stats = []
    bcopies = []
    for q in range(_RPW):
        p = p0 + q
        s_star = jnp.sum(jnp.where(iota == p, sv, 0))
        rbef = jnp.sum(jnp.where(iota == p, rv, 0))
        kq = jnp.sum(jnp.where(iota == p, kv, 0))
        s_c = jnp.maximum(s_star, 0)
        stats.append((s_star, rbef, kq, s_c))
        bcopies.append(pltpu.async_copy(
            words_w.at[pl.ds(s_c * _SUBW, _SUBW), :], sbufs[q], bsems[q]))

    for q in range(_RPW):
        p = p0 + q
        s_star, rbef, kq, s_c = stats[q]
        sbuf = sbufs[q]
        bcopies[q].wait()

        def cnt(a):
            fv = _fold_bytes(sbuf[a, pl.ds(goff, 16)])
            return jnp.sum(jnp.where(iota == p, fv, 0))

        def w_cond(c):
            _, run2, t = c
            return run2 + t <= kq

        def w_body(c):
            a, run2, t = c
            a2 = a + 1
            return (a2, run2 + t, cnt(a2))

        a, run2, _ = lax.while_loop(
            w_cond, w_body, (jnp.int32(0), rbef, cnt(jnp.int32(0))))

        vw = sbuf[a, pl.ds(goff, 16)]
        wsc = jnp.sum(jnp.where(iota == p, vw, 0))
        m8 = jnp.int32(0xFF)
        b0 = wsc & m8
        b1 = lax.shift_right_logical(wsc, 8) & m8
        b2 = lax.shift_right_logical(wsc, 16) & m8
        b3 = lax.shift_right_logical(wsc, 24) & m8
        rl = kq - run2
        e1 = b0
        e2 = b0 + b1
        e3 = e2 + b2
        i_sel = jnp.where(
            (rl == 0) & (b0 > 0), 0,
            jnp.where((e1 == rl) & (b1 > 0), 1,
                      jnp.where((e2 == rl) & (b2 > 0), 2,
                                jnp.where((e3 == rl) & (b3 > 0), 3, 0))))
        idx = (s_c * _SUBW + a) * 4 + i_sel
        idx = jnp.where(s_star < 0, 0, idx)
        res = jnp.where(iota == q, idx, res)

    resv[...] = res
    pltpu.sync_copy(resv, out_ref.at[pl.ds(wid * 16, 16)])


_mesh = plsc.VectorSubcoreMesh(core_axis_name="c", subcore_axis_name="s")

_sc_p1 = pl.kernel(
    _p1_body,
    out_type=jax.ShapeDtypeStruct((_META,), jnp.int32),
    mesh=_mesh,
    scratch_types=[
        pltpu.VMEM((_SUBW, 128), jnp.int32),
        pltpu.VMEM((_SUBW, 128), jnp.int32),
        pltpu.VMEM((_NSUB * 128,), jnp.int32),
        pltpu.SemaphoreType.DMA,
        pltpu.SemaphoreType.DMA,
    ],
    compiler_params=pltpu.CompilerParams(needs_layout_passes=False),
)

_sc_p2 = pl.kernel(
    _p2_body,
    out_type=(jax.ShapeDtypeStruct((_B * _CONT,), jnp.float32),
              jax.ShapeDtypeStruct((_NW * 16,), jnp.int32)),
    mesh=_mesh,
    scratch_types=[
        pltpu.VMEM((_S * 16,), jnp.int32),
        pltpu.VMEM((_SUBW, 128), jnp.int32),
        pltpu.VMEM((_SUBW, 128), jnp.int32),
        pltpu.VMEM((_SUBW, 128), jnp.int32),
        pltpu.VMEM((_SUBW, 128), jnp.int32),
        pltpu.VMEM((16,), jnp.float32),
        pltpu.VMEM((_RPW * _CONT,), jnp.float32),
        pltpu.VMEM((16,), jnp.int32),
        pltpu.SemaphoreType.DMA,
        pltpu.SemaphoreType.DMA,
        pltpu.SemaphoreType.DMA,
        pltpu.SemaphoreType.DMA,
        pltpu.SemaphoreType.DMA,
    ],
    compiler_params=pltpu.CompilerParams(needs_layout_passes=False),
)


def kernel(states, mask):
    del states
    key = jax.random.key(42)
    ka, kb = jax.random.split(key)
    u01 = jax.random.uniform(ka, (_B, _CONT), dtype=jnp.float32)
    u = jax.random.uniform(kb, (_B,), dtype=jnp.float32)

    mbytes = jnp.pad(mask.astype(jnp.uint8).T, ((0, _NVP - _NV), (0, 0)))

    meta = _sc_p1(mbytes)
    cont_flat, disc_flat = _sc_p2(mbytes, meta, u, u01.reshape(-1))
    cont = cont_flat.reshape(_B, _CONT)
    disc = disc_flat.reshape(_NW, 16)[:, :_RPW].reshape(_B)
    return cont, disc
